# baseline (speedup 1.0000x reference)
import jax
import jax.numpy as jnp
from jax import lax
from jax.experimental import pallas as pl
from jax.experimental.pallas import tpu as pltpu

_D = 64
_A = 16
_B = 16
_H = 32
_W = 32
_HW = _H * _W
_NTOK = _B * _HW
_CLW = 0.1
_PLW = 0.33
_BB = 4


def _avq_kernel(x_ref, cb_ref, cbt_ref, prev_ref, quant_ref, loss_ref):
    b = pl.program_id(0)
    nb = pl.num_programs(0)
    cb = cb_ref[...]
    cbt = cbt_ref[...]
    cbnorm = jnp.sum(cb * cb, axis=1, keepdims=True)

    part = jnp.zeros((1, 128), jnp.float32)
    lane = lax.broadcasted_iota(jnp.int32, (1, 128), 1)
    rowid = lax.broadcasted_iota(jnp.int32, (_A, _HW), 0)
    for sb in range(_BB):
        x = x_ref[sb].reshape(_D, _HW)
        dot = jax.lax.dot_general(cb, x, (((1,), (0,)), ((), ())),
                                  preferred_element_type=jnp.float32)
        xnorm = jnp.sum(x * x, axis=0, keepdims=True)
        dist = (xnorm + cbnorm) - 2.0 * dot

        for lvl in range(4):
            a = 2 ** (lvl + 1)
            big = jnp.where(rowid < a, dist, jnp.inf)
            minv = jnp.min(big, axis=0, keepdims=True)
            idx = jnp.min(jnp.where(big == minv, rowid, _A), axis=0,
                          keepdims=True)
            onehot = (rowid == idx).astype(jnp.float32)
            q = jax.lax.dot_general(cbt, onehot, (((1,), (0,)), ((), ())),
                                    preferred_element_type=jnp.float32)
            out = x + (q - x)
            quant_ref[lvl, sb] = out.reshape(_D, _H, _W)
            part = part + jnp.where(lane == lvl, jnp.sum(minv), 0.0)

    @pl.when(b == 0)
    def _init():
        loss_ref[...] = jnp.zeros((1, 128), jnp.float32)

    acc = loss_ref[...] + part

    m = acc * jnp.float32(1.0 / (_NTOK * _D))
    prev = prev_ref[...]
    dv = prev - cb
    dv2 = dv * dv
    prow = lax.broadcasted_iota(jnp.int32, (_A, _D), 0)
    p2 = jnp.sum(jnp.where(prow < 2, dv2, 0.0)) * jnp.float32(1.0 / (2 * _D))
    p4 = jnp.sum(jnp.where(prow < 4, dv2, 0.0)) * jnp.float32(1.0 / (4 * _D))
    p8 = jnp.sum(jnp.where(prow < 8, dv2, 0.0)) * jnp.float32(1.0 / (8 * _D))
    clw_vec = jnp.where(lane < 2, jnp.float32(_CLW), 0.0)
    prox_vec = (jnp.where(lane == 1, jnp.float32(1 * _PLW) * p2, 0.0)
                + jnp.where(lane == 2, jnp.float32(_PLW) * p4, 0.0)
                + jnp.where(lane == 3, jnp.float32(_PLW) * p8, 0.0))
    final = (m + clw_vec * m) + prox_vec

    loss_ref[...] = jnp.where(b == nb - 1, final, acc)


@jax.jit
def _run(x4, cb16, cbt16, prev):
    quant, loss = pl.pallas_call(
        _avq_kernel,
        grid=(_B // _BB,),
        in_specs=[
            pl.BlockSpec((_BB, _D, _H, _W), lambda b: (b, 0, 0, 0)),
            pl.BlockSpec((_A, _D), lambda b: (0, 0)),
            pl.BlockSpec((_D, _A), lambda b: (0, 0)),
            pl.BlockSpec((_A, _D), lambda b: (0, 0)),
        ],
        out_specs=[
            pl.BlockSpec((4, _BB, _D, _H, _W), lambda b: (0, b, 0, 0, 0)),
            pl.BlockSpec((1, 128), lambda b: (0, 0)),
        ],
        out_shape=[
            jax.ShapeDtypeStruct((4, _B, _D, _H, _W), jnp.float32),
            jax.ShapeDtypeStruct((1, 128), jnp.float32),
        ],
        compiler_params=pltpu.CompilerParams(
            dimension_semantics=("arbitrary",)),
    )(x4, cb16, cbt16, prev)
    return quant, loss


def kernel(input_data, num_active_vectors, previous_active_vectors, codebook):
    cb16 = codebook[:_A]
    quant, loss = _run(input_data, cb16, cb16.T, previous_active_vectors)
    losses = loss[0, :4]
    return quant, losses, cb16


# speedup vs baseline: 1.1707x; 1.1707x over previous
import functools

import jax
import jax.numpy as jnp
from jax import lax
from jax.experimental import pallas as pl
from jax.experimental.pallas import tpu as pltpu
from jax.experimental.pallas import tpu_sc as plsc

_D = 64
_A = 16
_B = 16
_HW = 1024
_NTOK = _B * _HW
_CLW = 0.1
_PLW = 0.33
_BB = 8

_NC = 2
_NS = 16
_TPW = _NTOK // (_NC * _NS)
_L = 16


def _dist_kernel(x_ref, cb_ref, prev_ref, idx_ref, loss_ref):
    b = pl.program_id(0)
    nb = pl.num_programs(0)
    cb = cb_ref[...]
    cbnorm = jnp.sum(cb * cb, axis=1, keepdims=True)

    part = jnp.zeros((1, 128), jnp.float32)
    lane = lax.broadcasted_iota(jnp.int32, (1, 128), 1)
    rowid = lax.broadcasted_iota(jnp.int32, (_A, _HW), 0)
    for sb in range(_BB):
        x = x_ref[sb]
        dot = jax.lax.dot_general(cb, x, (((1,), (0,)), ((), ())),
                                  preferred_element_type=jnp.float32)
        xnorm = jnp.sum(x * x, axis=0, keepdims=True)
        dist = (xnorm + cbnorm) - 2.0 * dot

        for lvl in range(4):
            a = 2 ** (lvl + 1)
            big = jnp.where(rowid < a, dist, jnp.inf)
            minv = jnp.min(big, axis=0, keepdims=True)
            idx = jnp.min(jnp.where(big == minv, rowid, _A), axis=0,
                          keepdims=True)
            idx_ref[lvl, sb] = idx[0]
            part = part + jnp.where(lane == lvl, jnp.sum(minv), 0.0)

    @pl.when(b == 0)
    def _init():
        loss_ref[...] = jnp.zeros((1, 128), jnp.float32)

    acc = loss_ref[...] + part

    m = acc * jnp.float32(1.0 / (_NTOK * _D))
    prev = prev_ref[...]
    dv = prev - cb
    dv2 = dv * dv
    prow = lax.broadcasted_iota(jnp.int32, (_A, _D), 0)
    p2 = jnp.sum(jnp.where(prow < 2, dv2, 0.0)) * jnp.float32(1.0 / (2 * _D))
    p4 = jnp.sum(jnp.where(prow < 4, dv2, 0.0)) * jnp.float32(1.0 / (4 * _D))
    p8 = jnp.sum(jnp.where(prow < 8, dv2, 0.0)) * jnp.float32(1.0 / (8 * _D))
    clw_vec = jnp.where(lane < 2, jnp.float32(_CLW), 0.0)
    prox_vec = (jnp.where(lane == 1, jnp.float32(1 * _PLW) * p2, 0.0)
                + jnp.where(lane == 2, jnp.float32(_PLW) * p4, 0.0)
                + jnp.where(lane == 3, jnp.float32(_PLW) * p8, 0.0))
    final = (m + clw_vec * m) + prox_vec

    loss_ref[...] = jnp.where(b == nb - 1, final, acc)


def _tc_stage(x3, cb16, prev):
    return pl.pallas_call(
        _dist_kernel,
        grid=(_B // _BB,),
        in_specs=[
            pl.BlockSpec((_BB, _D, _HW), lambda b: (b, 0, 0)),
            pl.BlockSpec((_A, _D), lambda b: (0, 0)),
            pl.BlockSpec((_A, _D), lambda b: (0, 0)),
        ],
        out_specs=[
            pl.BlockSpec((4, _BB, _HW), lambda b: (0, b, 0)),
            pl.BlockSpec((1, 128), lambda b: (0, 0)),
        ],
        out_shape=[
            jax.ShapeDtypeStruct((4, _B, _HW), jnp.int32),
            jax.ShapeDtypeStruct((1, 128), jnp.float32),
        ],
        compiler_params=pltpu.CompilerParams(
            dimension_semantics=("arbitrary",)),
    )(x3, cb16, prev)


def _sc_gather(idx_ref, cbtab_ref, quant_ref, idx_v, cbtab_v, obuf, sem0, sem1):
    wid = lax.axis_index("s") * _NC + lax.axis_index("c")
    b = wid // 2
    hw0 = (wid % 2) * _TPW

    pltpu.sync_copy(cbtab_ref, cbtab_v)
    pltpu.sync_copy(idx_ref.at[:, b, pl.ds(hw0, _TPW)], idx_v)

    lanes = lax.broadcasted_iota(jnp.int32, (_L,), 0)
    ngrp = _TPW // _L
    sems = (sem0, sem1)
    pending = [None, None]

    for lvl in range(4):
        bi = lvl % 2
        if pending[bi] is not None:
            pending[bi].wait()

        def body(g, carry, lvl=lvl, bi=bi):
            iv = idx_v[lvl, pl.ds(g * _L, _L)]
            ivs = iv * _L + lanes
            for c in range(_D):
                vals = plsc.load_gather(cbtab_v, [ivs + (c * (_A * _L))])
                obuf[bi, c, pl.ds(g * _L, _L)] = vals
            return carry

        lax.fori_loop(0, ngrp, body, None)
        pending[bi] = pltpu.async_copy(
            obuf.at[bi], quant_ref.at[lvl, b, :, pl.ds(hw0, _TPW)], sems[bi])
    pending[0].wait()
    pending[1].wait()


def _make_sc_stage():
    mesh = plsc.VectorSubcoreMesh(core_axis_name="c", subcore_axis_name="s")
    return functools.partial(
        pl.kernel,
        mesh=mesh,
        out_type=jax.ShapeDtypeStruct((4, _B, _D, _HW), jnp.float32),
        scratch_types=[
            pltpu.VMEM((4, _TPW), jnp.int32),
            pltpu.VMEM((_D * _A * _L,), jnp.float32),
            pltpu.VMEM((2, _D, _TPW), jnp.float32),
            pltpu.SemaphoreType.DMA,
            pltpu.SemaphoreType.DMA,
        ],
        compiler_params=pltpu.CompilerParams(needs_layout_passes=False),
    )(_sc_gather)


@jax.jit
def _run(x3, cb16, cbtab, prev):
    idx, loss = _tc_stage(x3, cb16, prev)
    quant = _make_sc_stage()(idx, cbtab)
    return quant, loss


def kernel(input_data, num_active_vectors, previous_active_vectors, codebook):
    x3 = input_data.reshape(_B, _D, _HW)
    cb16 = codebook[:_A]
    cbtab = jnp.broadcast_to(cb16.T[:, :, None], (_D, _A, _L)).reshape(-1)
    quant, loss = _run(x3, cb16, cbtab, previous_active_vectors)
    quantized = quant.reshape(4, _B, _D, 32, 32)
    losses = loss[0, :4]
    return quantized, losses, cb16


# speedup vs baseline: 1.4044x; 1.1996x over previous
import functools

import jax
import jax.numpy as jnp
from jax import lax
from jax.experimental import pallas as pl
from jax.experimental.pallas import tpu as pltpu
from jax.experimental.pallas import tpu_sc as plsc

_D = 64
_A = 16
_B = 16
_HW = 1024
_NTOK = _B * _HW
_CLW = 0.1
_PLW = 0.33
_BB = 8

_NC = 2
_NS = 16
_TPW = _NTOK // (_NC * _NS)
_L = 16


def _dist_kernel(x_ref, cb_ref, prev_ref, idx_ref, loss_ref):
    b = pl.program_id(0)
    nb = pl.num_programs(0)
    cb = cb_ref[...]
    cbnorm = jnp.sum(cb * cb, axis=1, keepdims=True)

    part = jnp.zeros((1, 128), jnp.float32)
    lane = lax.broadcasted_iota(jnp.int32, (1, 128), 1)
    rowid = lax.broadcasted_iota(jnp.int32, (_A, _HW), 0)
    for sb in range(_BB):
        x = x_ref[sb]
        dot = jax.lax.dot_general(cb, x, (((1,), (0,)), ((), ())),
                                  preferred_element_type=jnp.float32)
        xnorm = jnp.sum(x * x, axis=0, keepdims=True)
        dist = (xnorm + cbnorm) - 2.0 * dot

        for lvl in range(4):
            a = 2 ** (lvl + 1)
            big = jnp.where(rowid < a, dist, jnp.inf)
            minv = jnp.min(big, axis=0, keepdims=True)
            idx = jnp.min(jnp.where(big == minv, rowid, _A), axis=0,
                          keepdims=True)
            idx_ref[lvl, sb] = idx[0]
            part = part + jnp.where(lane == lvl, jnp.sum(minv), 0.0)

    @pl.when(b == 0)
    def _init():
        loss_ref[...] = jnp.zeros((1, 128), jnp.float32)

    acc = loss_ref[...] + part

    m = acc * jnp.float32(1.0 / (_NTOK * _D))
    prev = prev_ref[...]
    dv = prev - cb
    dv2 = dv * dv
    prow = lax.broadcasted_iota(jnp.int32, (_A, _D), 0)
    p2 = jnp.sum(jnp.where(prow < 2, dv2, 0.0)) * jnp.float32(1.0 / (2 * _D))
    p4 = jnp.sum(jnp.where(prow < 4, dv2, 0.0)) * jnp.float32(1.0 / (4 * _D))
    p8 = jnp.sum(jnp.where(prow < 8, dv2, 0.0)) * jnp.float32(1.0 / (8 * _D))
    clw_vec = jnp.where(lane < 2, jnp.float32(_CLW), 0.0)
    prox_vec = (jnp.where(lane == 1, jnp.float32(1 * _PLW) * p2, 0.0)
                + jnp.where(lane == 2, jnp.float32(_PLW) * p4, 0.0)
                + jnp.where(lane == 3, jnp.float32(_PLW) * p8, 0.0))
    final = (m + clw_vec * m) + prox_vec

    loss_ref[...] = jnp.where(b == nb - 1, final, acc)


def _tc_stage(x3, cb16, prev):
    return pl.pallas_call(
        _dist_kernel,
        grid=(_B // _BB,),
        in_specs=[
            pl.BlockSpec((_BB, _D, _HW), lambda b: (b, 0, 0)),
            pl.BlockSpec((_A, _D), lambda b: (0, 0)),
            pl.BlockSpec((_A, _D), lambda b: (0, 0)),
        ],
        out_specs=[
            pl.BlockSpec((4, _BB, _HW), lambda b: (0, b, 0)),
            pl.BlockSpec((1, 128), lambda b: (0, 0)),
        ],
        out_shape=[
            jax.ShapeDtypeStruct((4, _B, _HW), jnp.int32),
            jax.ShapeDtypeStruct((1, 128), jnp.float32),
        ],
        compiler_params=pltpu.CompilerParams(
            dimension_semantics=("arbitrary",)),
    )(x3, cb16, prev)


def _sc_gather(idx_ref, cbtab_ref, quant_ref, idx_v, cbtab_v, obuf, sem0, sem1):
    wid = lax.axis_index("s") * _NC + lax.axis_index("c")
    b = wid // 2
    hw0 = (wid % 2) * _TPW

    pltpu.sync_copy(cbtab_ref, cbtab_v)
    pltpu.sync_copy(idx_ref.at[:, b, pl.ds(hw0, _TPW)], idx_v)

    lanes = lax.broadcasted_iota(jnp.int32, (_L,), 0)
    ngrp = _TPW // _L
    sems = (sem0, sem1)
    pending = [None, None]

    for lvl in range(4):
        bi = lvl % 2
        if pending[bi] is not None:
            pending[bi].wait()

        @plsc.parallel_loop(0, ngrp, unroll=1)
        def body(g, lvl=lvl, bi=bi):
            iv = idx_v[lvl, pl.ds(g * _L, _L)]
            ivs = iv * _L + lanes
            for c in range(_D):
                vals = plsc.load_gather(cbtab_v, [ivs + (c * (_A * _L))])
                obuf[bi, c, pl.ds(g * _L, _L)] = vals
        pending[bi] = pltpu.async_copy(
            obuf.at[bi], quant_ref.at[lvl, b, :, pl.ds(hw0, _TPW)], sems[bi])
    pending[0].wait()
    pending[1].wait()


def _make_sc_stage():
    mesh = plsc.VectorSubcoreMesh(core_axis_name="c", subcore_axis_name="s")
    return functools.partial(
        pl.kernel,
        mesh=mesh,
        out_type=jax.ShapeDtypeStruct((4, _B, _D, _HW), jnp.float32),
        scratch_types=[
            pltpu.VMEM((4, _TPW), jnp.int32),
            pltpu.VMEM((_D * _A * _L,), jnp.float32),
            pltpu.VMEM((2, _D, _TPW), jnp.float32),
            pltpu.SemaphoreType.DMA,
            pltpu.SemaphoreType.DMA,
        ],
        compiler_params=pltpu.CompilerParams(needs_layout_passes=False),
    )(_sc_gather)


@jax.jit
def _run(x3, cb16, cbtab, prev):
    idx, loss = _tc_stage(x3, cb16, prev)
    quant = _make_sc_stage()(idx, cbtab)
    return quant, loss


def kernel(input_data, num_active_vectors, previous_active_vectors, codebook):
    x3 = input_data.reshape(_B, _D, _HW)
    cb16 = codebook[:_A]
    cbtab = jnp.broadcast_to(cb16.T[:, :, None], (_D, _A, _L)).reshape(-1)
    quant, loss = _run(x3, cb16, cbtab, previous_active_vectors)
    quantized = quant.reshape(4, _B, _D, 32, 32)
    losses = loss[0, :4]
    return quantized, losses, cb16


# speedup vs baseline: 1.5020x; 1.0696x over previous
import functools

import jax
import jax.numpy as jnp
from jax import lax
from jax.experimental import pallas as pl
from jax.experimental.pallas import tpu as pltpu
from jax.experimental.pallas import tpu_sc as plsc

_D = 64
_A = 16
_B = 16
_HW = 1024
_NTOK = _B * _HW
_CLW = 0.1
_PLW = 0.33
_BB = 8

_NC = 2
_NS = 16
_TPW = _NTOK // (_NC * _NS)
_L = 16


def _dist_kernel(x_ref, cb_ref, prev_ref, idx_ref, loss_ref):
    b = pl.program_id(0)
    nb = pl.num_programs(0)
    cb = cb_ref[...]
    cbnorm = jnp.sum(cb * cb, axis=1, keepdims=True)

    part = jnp.zeros((1, 128), jnp.float32)
    lane = lax.broadcasted_iota(jnp.int32, (1, 128), 1)
    rowid = lax.broadcasted_iota(jnp.int32, (_A, _HW), 0)
    for sb in range(_BB):
        x = x_ref[sb]
        dot = jax.lax.dot_general(cb, x, (((1,), (0,)), ((), ())),
                                  preferred_element_type=jnp.float32)
        xnorm = jnp.sum(x * x, axis=0, keepdims=True)
        dist = (xnorm + cbnorm) - 2.0 * dot

        for lvl in range(4):
            a = 2 ** (lvl + 1)
            big = jnp.where(rowid < a, dist, jnp.inf)
            minv = jnp.min(big, axis=0, keepdims=True)
            idx = jnp.min(jnp.where(big == minv, rowid, _A), axis=0,
                          keepdims=True)
            idx_ref[lvl, sb] = idx[0]
            part = part + jnp.where(lane == lvl, jnp.sum(minv), 0.0)

    @pl.when(b == 0)
    def _init():
        loss_ref[...] = jnp.zeros((1, 128), jnp.float32)

    acc = loss_ref[...] + part

    m = acc * jnp.float32(1.0 / (_NTOK * _D))
    prev = prev_ref[...]
    dv = prev - cb
    dv2 = dv * dv
    prow = lax.broadcasted_iota(jnp.int32, (_A, _D), 0)
    p2 = jnp.sum(jnp.where(prow < 2, dv2, 0.0)) * jnp.float32(1.0 / (2 * _D))
    p4 = jnp.sum(jnp.where(prow < 4, dv2, 0.0)) * jnp.float32(1.0 / (4 * _D))
    p8 = jnp.sum(jnp.where(prow < 8, dv2, 0.0)) * jnp.float32(1.0 / (8 * _D))
    clw_vec = jnp.where(lane < 2, jnp.float32(_CLW), 0.0)
    prox_vec = (jnp.where(lane == 1, jnp.float32(1 * _PLW) * p2, 0.0)
                + jnp.where(lane == 2, jnp.float32(_PLW) * p4, 0.0)
                + jnp.where(lane == 3, jnp.float32(_PLW) * p8, 0.0))
    final = (m + clw_vec * m) + prox_vec

    loss_ref[...] = jnp.where(b == nb - 1, final, acc)


def _tc_stage(x3, cb16, prev):
    return pl.pallas_call(
        _dist_kernel,
        grid=(_B // _BB,),
        in_specs=[
            pl.BlockSpec((_BB, _D, _HW), lambda b: (b, 0, 0)),
            pl.BlockSpec((_A, _D), lambda b: (0, 0)),
            pl.BlockSpec((_A, _D), lambda b: (0, 0)),
        ],
        out_specs=[
            pl.BlockSpec((4, _BB, _HW), lambda b: (0, b, 0)),
            pl.BlockSpec((1, 128), lambda b: (0, 0)),
        ],
        out_shape=[
            jax.ShapeDtypeStruct((4, _B, _HW), jnp.int32),
            jax.ShapeDtypeStruct((1, 128), jnp.float32),
        ],
        compiler_params=pltpu.CompilerParams(
            dimension_semantics=("arbitrary",)),
    )(x3, cb16, prev)


def _sc_gather(idx_ref, cbtab_ref, quant_ref, idx_v, cbtab_v, obuf, sem0, sem1):
    wid = lax.axis_index("s") * _NC + lax.axis_index("c")
    b = wid // 2
    hw0 = (wid % 2) * _TPW

    pltpu.sync_copy(cbtab_ref, cbtab_v)
    pltpu.sync_copy(idx_ref.at[:, b, pl.ds(hw0, _TPW)], idx_v)

    lanes = lax.broadcasted_iota(jnp.int32, (_L,), 0)
    ngrp = _TPW // _L
    sems = (sem0, sem1)
    pending = [None, None]

    for lvl in range(4):
        bi = lvl % 2
        if pending[bi] is not None:
            pending[bi].wait()

        @plsc.parallel_loop(0, ngrp, unroll=1)
        def body(g, lvl=lvl, bi=bi):
            iv = idx_v[lvl, pl.ds(g * _L, _L)]
            ivs = iv * _L + lanes
            for c in range(_D):
                vals = plsc.load_gather(
                    cbtab_v.at[pl.ds(c * (_A * _L), _A * _L)], [ivs])
                obuf[bi, c, pl.ds(g * _L, _L)] = vals
        pending[bi] = pltpu.async_copy(
            obuf.at[bi], quant_ref.at[lvl, b, :, pl.ds(hw0, _TPW)], sems[bi])
    pending[0].wait()
    pending[1].wait()


def _make_sc_stage():
    mesh = plsc.VectorSubcoreMesh(core_axis_name="c", subcore_axis_name="s")
    return functools.partial(
        pl.kernel,
        mesh=mesh,
        out_type=jax.ShapeDtypeStruct((4, _B, _D, _HW), jnp.float32),
        scratch_types=[
            pltpu.VMEM((4, _TPW), jnp.int32),
            pltpu.VMEM((_D * _A * _L,), jnp.float32),
            pltpu.VMEM((2, _D, _TPW), jnp.float32),
            pltpu.SemaphoreType.DMA,
            pltpu.SemaphoreType.DMA,
        ],
        compiler_params=pltpu.CompilerParams(needs_layout_passes=False),
    )(_sc_gather)


@jax.jit
def _run(x3, cb16, cbtab, prev):
    idx, loss = _tc_stage(x3, cb16, prev)
    quant = _make_sc_stage()(idx, cbtab)
    return quant, loss


def kernel(input_data, num_active_vectors, previous_active_vectors, codebook):
    x3 = input_data.reshape(_B, _D, _HW)
    cb16 = codebook[:_A]
    cbtab = jnp.broadcast_to(cb16.T[:, :, None], (_D, _A, _L)).reshape(-1)
    quant, loss = _run(x3, cb16, cbtab, previous_active_vectors)
    quantized = quant.reshape(4, _B, _D, 32, 32)
    losses = loss[0, :4]
    return quantized, losses, cb16


# speedup vs baseline: 1.5579x; 1.0372x over previous
import functools

import jax
import jax.numpy as jnp
from jax import lax
from jax.experimental import pallas as pl
from jax.experimental.pallas import tpu as pltpu
from jax.experimental.pallas import tpu_sc as plsc

_D = 64
_A = 16
_B = 16
_HW = 1024
_NTOK = _B * _HW
_CLW = 0.1
_PLW = 0.33
_BB = 8

_NC = 2
_NS = 16
_TPW = _NTOK // (_NC * _NS)
_L = 16


def _dist_kernel(x_ref, cb_ref, prev_ref, idx_ref, loss_ref):
    b = pl.program_id(0)
    nb = pl.num_programs(0)
    cb = cb_ref[...]
    cbnorm = jnp.sum(cb * cb, axis=1, keepdims=True)

    part = jnp.zeros((1, 128), jnp.float32)
    lane = lax.broadcasted_iota(jnp.int32, (1, 128), 1)
    rowid = lax.broadcasted_iota(jnp.int32, (_A, _HW), 0)
    for sb in range(_BB):
        x = x_ref[sb]
        dot = jax.lax.dot_general(cb, x, (((1,), (0,)), ((), ())),
                                  preferred_element_type=jnp.float32)
        xnorm = jnp.sum(x * x, axis=0, keepdims=True)
        dist = (xnorm + cbnorm) - 2.0 * dot

        for lvl in range(4):
            a = 2 ** (lvl + 1)
            big = jnp.where(rowid < a, dist, jnp.inf)
            minv = jnp.min(big, axis=0, keepdims=True)
            idx = jnp.min(jnp.where(big == minv, rowid, _A), axis=0,
                          keepdims=True)
            idx_ref[lvl, sb] = idx[0]
            part = part + jnp.where(lane == lvl, jnp.sum(minv), 0.0)

    @pl.when(b == 0)
    def _init():
        loss_ref[...] = jnp.zeros((1, 128), jnp.float32)

    acc = loss_ref[...] + part

    m = acc * jnp.float32(1.0 / (_NTOK * _D))
    prev = prev_ref[...]
    dv = prev - cb
    dv2 = dv * dv
    prow = lax.broadcasted_iota(jnp.int32, (_A, _D), 0)
    p2 = jnp.sum(jnp.where(prow < 2, dv2, 0.0)) * jnp.float32(1.0 / (2 * _D))
    p4 = jnp.sum(jnp.where(prow < 4, dv2, 0.0)) * jnp.float32(1.0 / (4 * _D))
    p8 = jnp.sum(jnp.where(prow < 8, dv2, 0.0)) * jnp.float32(1.0 / (8 * _D))
    clw_vec = jnp.where(lane < 2, jnp.float32(_CLW), 0.0)
    prox_vec = (jnp.where(lane == 1, jnp.float32(1 * _PLW) * p2, 0.0)
                + jnp.where(lane == 2, jnp.float32(_PLW) * p4, 0.0)
                + jnp.where(lane == 3, jnp.float32(_PLW) * p8, 0.0))
    final = (m + clw_vec * m) + prox_vec

    loss_ref[...] = jnp.where(b == nb - 1, final, acc)


def _tc_stage(x3, cb16, prev):
    return pl.pallas_call(
        _dist_kernel,
        grid=(_B // _BB,),
        in_specs=[
            pl.BlockSpec((_BB, _D, _HW), lambda b: (b, 0, 0)),
            pl.BlockSpec((_A, _D), lambda b: (0, 0)),
            pl.BlockSpec((_A, _D), lambda b: (0, 0)),
        ],
        out_specs=[
            pl.BlockSpec((4, _BB, _HW), lambda b: (0, b, 0)),
            pl.BlockSpec((1, 128), lambda b: (0, 0)),
        ],
        out_shape=[
            jax.ShapeDtypeStruct((4, _B, _HW), jnp.int32),
            jax.ShapeDtypeStruct((1, 128), jnp.float32),
        ],
        compiler_params=pltpu.CompilerParams(
            dimension_semantics=("arbitrary",)),
    )(x3, cb16, prev)


def _sc_gather(idx_ref, cbtab_ref, quant_ref, idx_v, cbtab_v, obuf, sem0, sem1):
    wid = lax.axis_index("s") * _NC + lax.axis_index("c")
    b = wid // 2
    hw0 = (wid % 2) * _TPW

    pltpu.sync_copy(cbtab_ref, cbtab_v)
    pltpu.sync_copy(idx_ref.at[:, b, pl.ds(hw0, _TPW)], idx_v)

    lanes = lax.broadcasted_iota(jnp.int32, (_L,), 0)
    ngrp = _TPW // _L
    sems = (sem0, sem1)
    pending = [None, None]

    for lvl in range(4):
        bi = lvl % 2
        if pending[bi] is not None:
            pending[bi].wait()

        @plsc.parallel_loop(0, ngrp, unroll=1)
        def body(g, lvl=lvl, bi=bi):
            iv = idx_v[lvl, pl.ds(g * _L, _L)]
            ivs = iv * _L + lanes

            @plsc.parallel_loop(0, _D, unroll=8)
            def cbody(c, ivs=ivs, g=g, bi=bi):
                vals = plsc.load_gather(
                    cbtab_v.at[pl.ds(c * (_A * _L), _A * _L)], [ivs])
                obuf[bi, c, pl.ds(g * _L, _L)] = vals
        pending[bi] = pltpu.async_copy(
            obuf.at[bi], quant_ref.at[lvl, b, :, pl.ds(hw0, _TPW)], sems[bi])
    pending[0].wait()
    pending[1].wait()


def _make_sc_stage():
    mesh = plsc.VectorSubcoreMesh(core_axis_name="c", subcore_axis_name="s")
    return functools.partial(
        pl.kernel,
        mesh=mesh,
        out_type=jax.ShapeDtypeStruct((4, _B, _D, _HW), jnp.float32),
        scratch_types=[
            pltpu.VMEM((4, _TPW), jnp.int32),
            pltpu.VMEM((_D * _A * _L,), jnp.float32),
            pltpu.VMEM((2, _D, _TPW), jnp.float32),
            pltpu.SemaphoreType.DMA,
            pltpu.SemaphoreType.DMA,
        ],
        compiler_params=pltpu.CompilerParams(needs_layout_passes=False),
    )(_sc_gather)


@jax.jit
def _run(x3, cb16, cbtab, prev):
    idx, loss = _tc_stage(x3, cb16, prev)
    quant = _make_sc_stage()(idx, cbtab)
    return quant, loss


def kernel(input_data, num_active_vectors, previous_active_vectors, codebook):
    x3 = input_data.reshape(_B, _D, _HW)
    cb16 = codebook[:_A]
    cbtab = jnp.broadcast_to(cb16.T[:, :, None], (_D, _A, _L)).reshape(-1)
    quant, loss = _run(x3, cb16, cbtab, previous_active_vectors)
    quantized = quant.reshape(4, _B, _D, 32, 32)
    losses = loss[0, :4]
    return quantized, losses, cb16


# speedup vs baseline: 1.5638x; 1.0038x over previous
import functools

import jax
import jax.numpy as jnp
from jax import lax
from jax.experimental import pallas as pl
from jax.experimental.pallas import tpu as pltpu
from jax.experimental.pallas import tpu_sc as plsc

_D = 64
_A = 16
_B = 16
_HW = 1024
_NTOK = _B * _HW
_CLW = 0.1
_PLW = 0.33
_BB = 8

_NC = 2
_NS = 16
_TPW = _NTOK // (_NC * _NS)
_L = 16


def _dist_kernel(x_ref, cb_ref, prev_ref, idx_ref, loss_ref):
    b = pl.program_id(0)
    nb = pl.num_programs(0)
    cb = cb_ref[...]
    cbnorm = jnp.sum(cb * cb, axis=1, keepdims=True)

    part = jnp.zeros((1, 128), jnp.float32)
    lane = lax.broadcasted_iota(jnp.int32, (1, 128), 1)
    rowid = lax.broadcasted_iota(jnp.int32, (_A, _HW), 0)
    for sb in range(_BB):
        x = x_ref[sb]
        dot = jax.lax.dot_general(cb, x, (((1,), (0,)), ((), ())),
                                  preferred_element_type=jnp.float32)
        xnorm = jnp.sum(x * x, axis=0, keepdims=True)
        dist = (xnorm + cbnorm) - 2.0 * dot

        for lvl in range(4):
            a = 2 ** (lvl + 1)
            big = jnp.where(rowid < a, dist, jnp.inf)
            minv = jnp.min(big, axis=0, keepdims=True)
            idx = jnp.min(jnp.where(big == minv, rowid, _A), axis=0,
                          keepdims=True)
            idx_ref[lvl, sb] = idx[0]
            part = part + jnp.where(lane == lvl, jnp.sum(minv), 0.0)

    @pl.when(b == 0)
    def _init():
        loss_ref[...] = jnp.zeros((1, 128), jnp.float32)

    acc = loss_ref[...] + part

    m = acc * jnp.float32(1.0 / (_NTOK * _D))
    prev = prev_ref[...]
    dv = prev - cb
    dv2 = dv * dv
    prow = lax.broadcasted_iota(jnp.int32, (_A, _D), 0)
    p2 = jnp.sum(jnp.where(prow < 2, dv2, 0.0)) * jnp.float32(1.0 / (2 * _D))
    p4 = jnp.sum(jnp.where(prow < 4, dv2, 0.0)) * jnp.float32(1.0 / (4 * _D))
    p8 = jnp.sum(jnp.where(prow < 8, dv2, 0.0)) * jnp.float32(1.0 / (8 * _D))
    clw_vec = jnp.where(lane < 2, jnp.float32(_CLW), 0.0)
    prox_vec = (jnp.where(lane == 1, jnp.float32(1 * _PLW) * p2, 0.0)
                + jnp.where(lane == 2, jnp.float32(_PLW) * p4, 0.0)
                + jnp.where(lane == 3, jnp.float32(_PLW) * p8, 0.0))
    final = (m + clw_vec * m) + prox_vec

    loss_ref[...] = jnp.where(b == nb - 1, final, acc)


def _tc_stage(x3, cb16, prev):
    return pl.pallas_call(
        _dist_kernel,
        grid=(_B // _BB,),
        in_specs=[
            pl.BlockSpec((_BB, _D, _HW), lambda b: (b, 0, 0)),
            pl.BlockSpec((_A, _D), lambda b: (0, 0)),
            pl.BlockSpec((_A, _D), lambda b: (0, 0)),
        ],
        out_specs=[
            pl.BlockSpec((4, _BB, _HW), lambda b: (0, b, 0)),
            pl.BlockSpec((1, 128), lambda b: (0, 0)),
        ],
        out_shape=[
            jax.ShapeDtypeStruct((4, _B, _HW), jnp.int32),
            jax.ShapeDtypeStruct((1, 128), jnp.float32),
        ],
        compiler_params=pltpu.CompilerParams(
            dimension_semantics=("arbitrary",)),
    )(x3, cb16, prev)


def _sc_gather(idx_ref, cbtab_ref, quant_ref, idx_v, cbtab_v, obuf, sem0, sem1):
    wid = lax.axis_index("s") * _NC + lax.axis_index("c")
    b = wid // 2
    hw0 = (wid % 2) * _TPW

    pltpu.sync_copy(cbtab_ref, cbtab_v)
    pltpu.sync_copy(idx_ref.at[:, b, pl.ds(hw0, _TPW)], idx_v)

    lanes = lax.broadcasted_iota(jnp.int32, (_L,), 0)
    ngrp = _TPW // _L
    sems = (sem0, sem1)
    pending = [None, None]

    for lvl in range(4):
        bi = lvl % 2
        if pending[bi] is not None:
            pending[bi].wait()

        @plsc.parallel_loop(0, ngrp, unroll=2)
        def body(g, lvl=lvl, bi=bi):
            iv = idx_v[lvl, pl.ds(g * _L, _L)]
            ivs = iv * _L + lanes

            @plsc.parallel_loop(0, _D, unroll=8)
            def cbody(c, ivs=ivs, g=g, bi=bi):
                vals = plsc.load_gather(
                    cbtab_v.at[pl.ds(c * (_A * _L), _A * _L)], [ivs])
                obuf[bi, c, pl.ds(g * _L, _L)] = vals
        pending[bi] = pltpu.async_copy(
            obuf.at[bi], quant_ref.at[lvl, b, :, pl.ds(hw0, _TPW)], sems[bi])
    pending[0].wait()
    pending[1].wait()


def _make_sc_stage():
    mesh = plsc.VectorSubcoreMesh(core_axis_name="c", subcore_axis_name="s")
    return functools.partial(
        pl.kernel,
        mesh=mesh,
        out_type=jax.ShapeDtypeStruct((4, _B, _D, _HW), jnp.float32),
        scratch_types=[
            pltpu.VMEM((4, _TPW), jnp.int32),
            pltpu.VMEM((_D * _A * _L,), jnp.float32),
            pltpu.VMEM((2, _D, _TPW), jnp.float32),
            pltpu.SemaphoreType.DMA,
            pltpu.SemaphoreType.DMA,
        ],
        compiler_params=pltpu.CompilerParams(needs_layout_passes=False),
    )(_sc_gather)


@jax.jit
def _run(x3, cb16, cbtab, prev):
    idx, loss = _tc_stage(x3, cb16, prev)
    quant = _make_sc_stage()(idx, cbtab)
    return quant, loss


def kernel(input_data, num_active_vectors, previous_active_vectors, codebook):
    x3 = input_data.reshape(_B, _D, _HW)
    cb16 = codebook[:_A]
    cbtab = jnp.broadcast_to(cb16.T[:, :, None], (_D, _A, _L)).reshape(-1)
    quant, loss = _run(x3, cb16, cbtab, previous_active_vectors)
    quantized = quant.reshape(4, _B, _D, 32, 32)
    losses = loss[0, :4]
    return quantized, losses, cb16
